# baseline (device time: 116402 ns/iter reference)
import jax
import jax.numpy as jnp
from jax import lax
from jax.experimental import pallas as pl
from jax.experimental.pallas import tpu as pltpu

N_DEV = 4
CH = 16
CORR_T = 64


def kernel(x, A, B, C):
    Bb, S, D = x.shape
    N = A.shape[1]
    BN = Bb * N
    n_chunks = S // CH

    dAT2 = jnp.tile(jnp.exp(A).T, (Bb, 1))
    BT3 = (B.transpose(0, 2, 1).reshape(BN, S)
           .reshape(BN, n_chunks, CH).transpose(1, 0, 2)
           .astype(jnp.bfloat16))
    C3 = (C.transpose(1, 0, 2).reshape(S, BN)
          .reshape(n_chunks, CH, BN).astype(jnp.bfloat16))

    def body(x_ref, dAT2_ref, BT3_ref, C3_ref, out_ref,
             x16_ref, send_ref, recv_ref, send_sem, recv_sem):
        my = lax.axis_index("i")
        right = (my + 1) % N_DEV

        dAT2_f32 = dAT2_ref[...]
        dAT2_v = dAT2_f32.astype(jnp.bfloat16)

        bi = lax.broadcasted_iota(jnp.int32, (Bb, BN), 0)
        bni = lax.broadcasted_iota(jnp.int32, (Bb, BN), 1)
        mask8 = (bni // N == bi).astype(jnp.bfloat16)
        bi_t = lax.broadcasted_iota(jnp.int32, (BN, Bb), 1)
        bni_t = lax.broadcasted_iota(jnp.int32, (BN, Bb), 0)
        mask8T = (bni_t // N == bi_t).astype(jnp.bfloat16)

        def cvt(c, _):
            sl = pl.ds(c * CH, CH)
            x16_ref[:, sl, :] = x_ref[:, sl, :].astype(jnp.bfloat16)
            return 0
        lax.fori_loop(0, n_chunks, cvt, 0)

        def chunk(c, h):
            t0 = c * CH
            xc = x16_ref[:, pl.ds(t0, CH), :]
            BTc = BT3_ref[c]
            Cc = C3_ref[c]
            ys = []
            for j in range(CH):
                xj = xc[:, j, :]
                W2 = mask8T * BTc[:, j][:, None]
                p = lax.dot_general(
                    W2, xj, (((1,), (0,)), ((), ())),
                    preferred_element_type=jnp.float32)
                h = h * dAT2_v + p.astype(jnp.bfloat16)
                W = mask8 * Cc[j, :][None, :]
                ys.append(lax.dot_general(
                    W, h, (((1,), (0,)), ((), ())),
                    preferred_element_type=jnp.float32))
            out_ref[:, pl.ds(t0, CH), :] = jnp.stack(ys, axis=1)
            return h

        h0 = jnp.zeros((BN, D), jnp.bfloat16)
        h_final = lax.fori_loop(0, n_chunks, chunk, h0)

        shift = pltpu.make_async_remote_copy(
            src_ref=send_ref, dst_ref=recv_ref,
            send_sem=send_sem, recv_sem=recv_sem,
            device_id=(right,), device_id_type=pl.DeviceIdType.MESH,
        )

        @pl.when(my < N_DEV - 1)
        def _():
            send_ref[...] = h_final
            shift.start()
            shift.wait_send()

        @pl.when(my > 0)
        def _():
            shift.wait_recv()

        carry = jnp.where(
            my == 0, 0.0, recv_ref[...].astype(jnp.float32))
        mask8_f32 = mask8.astype(jnp.float32)

        def corr(c, g):
            t0 = c * CH
            Cc = C3_ref[c]
            ys = []
            for j in range(CH):
                g = g * dAT2_f32
                W = mask8_f32 * Cc[j, :][None, :].astype(jnp.float32)
                ys.append(lax.dot_general(
                    W, g, (((1,), (0,)), ((), ())),
                    preferred_element_type=jnp.float32))
            out_ref[:, pl.ds(t0, CH), :] += jnp.stack(ys, axis=1)
            return g

        lax.fori_loop(0, CORR_T // CH, corr, carry)

    return pl.pallas_call(
        body,
        out_shape=jax.ShapeDtypeStruct((Bb, S, D), jnp.float32),
        in_specs=[pl.BlockSpec(memory_space=pltpu.VMEM)] * 4,
        out_specs=pl.BlockSpec(memory_space=pltpu.VMEM),
        scratch_shapes=[
            pltpu.VMEM((Bb, S, D), jnp.bfloat16),
            pltpu.VMEM((BN, D), jnp.bfloat16),
            pltpu.VMEM((BN, D), jnp.bfloat16),
            pltpu.SemaphoreType.DMA,
            pltpu.SemaphoreType.DMA,
        ],
    )(x, dAT2, BT3, C3)


# device time: 67247 ns/iter; 1.7310x vs baseline; 1.7310x over previous
import jax
import jax.numpy as jnp
from jax import lax
from jax.experimental import pallas as pl
from jax.experimental.pallas import tpu as pltpu

N_DEV = 4
CH = 16
CORR_T = 64


def kernel(x, A, B, C):
    Bb, S, D = x.shape
    N = A.shape[1]
    n_chunks = S // CH

    dAT = jnp.exp(A).T
    B4 = (B.transpose(0, 2, 1).reshape(Bb, N, n_chunks, CH)
          .transpose(2, 0, 1, 3).astype(jnp.bfloat16))
    C4 = (C.transpose(0, 2, 1).reshape(Bb, N, n_chunks, CH)
          .transpose(2, 0, 1, 3).astype(jnp.bfloat16))

    def body(x_ref, dAT_ref, B_ref, C_ref, out_ref,
             x16_ref, send_ref, recv_ref, send_sem, recv_sem):
        my = lax.axis_index("i")
        left = (my - 1) % N_DEV
        right = (my + 1) % N_DEV

        dAT_f32 = dAT_ref[...]
        dAT_v = dAT_f32.astype(jnp.bfloat16)

        def cvt(c, _):
            sl = pl.ds(c * CH, CH)
            x16_ref[:, sl, :] = x_ref[:, sl, :].astype(jnp.bfloat16)
            return 0
        lax.fori_loop(0, n_chunks, cvt, 0)

        def chunk(c, h):
            t0 = c * CH
            xc = x16_ref[:, pl.ds(t0, CH), :]
            Bc = B_ref[c]
            Cc = C_ref[c]
            ys = []
            for j in range(CH):
                xj = xc[:, j, :][:, None, :]
                bj = Bc[:, :, j][:, :, None]
                cj = Cc[:, :, j][:, :, None]
                h = h * dAT_v[None] + xj * bj
                ys.append(jnp.sum(h * cj, axis=1))
            out_ref[:, pl.ds(t0, CH), :] = (
                jnp.stack(ys, axis=1).astype(jnp.float32))
            return h

        h0 = jnp.zeros((Bb, N, D), jnp.bfloat16)
        h_final = lax.fori_loop(0, n_chunks, chunk, h0)

        shift = pltpu.make_async_remote_copy(
            src_ref=send_ref, dst_ref=recv_ref,
            send_sem=send_sem, recv_sem=recv_sem,
            device_id=(right,), device_id_type=pl.DeviceIdType.MESH,
        )

        @pl.when(my < N_DEV - 1)
        def _():
            send_ref[...] = h_final
            shift.start()
            shift.wait_send()

        @pl.when(my > 0)
        def _():
            shift.wait_recv()

        carry = jnp.where(
            my == 0, 0.0, recv_ref[...].astype(jnp.float32))

        def corr(c, g):
            t0 = c * CH
            Cc = C_ref[c]
            ys = []
            for j in range(CH):
                g = g * dAT_f32[None]
                cj = Cc[:, :, j][:, :, None].astype(jnp.float32)
                ys.append(jnp.sum(g * cj, axis=1))
            out_ref[:, pl.ds(t0, CH), :] += jnp.stack(ys, axis=1)
            return g

        lax.fori_loop(0, CORR_T // CH, corr, carry)

    return pl.pallas_call(
        body,
        out_shape=jax.ShapeDtypeStruct((Bb, S, D), jnp.float32),
        in_specs=[pl.BlockSpec(memory_space=pltpu.VMEM)] * 4,
        out_specs=pl.BlockSpec(memory_space=pltpu.VMEM),
        scratch_shapes=[
            pltpu.VMEM((Bb, S, D), jnp.bfloat16),
            pltpu.VMEM((Bb, N, D), jnp.bfloat16),
            pltpu.VMEM((Bb, N, D), jnp.bfloat16),
            pltpu.SemaphoreType.DMA,
            pltpu.SemaphoreType.DMA,
        ],
    )(x, dAT, B4, C4)
